# baseline (speedup 1.0000x reference)
import functools

import jax
import jax.numpy as jnp
from jax import lax
from jax.experimental import pallas as pl
from jax.experimental.pallas import tpu as pltpu
from jax.experimental.pallas import tpu_sc as plsc

_VOCAB = 1000000
_MAXLEN = 200
_DIM = 32
_BATCH = 4096
_NW = 32
_BBLK = _BATCH // _NW

_mesh = plsc.VectorSubcoreMesh(core_axis_name="c", subcore_axis_name="s")


@functools.partial(
    pl.kernel,
    out_type=jax.ShapeDtypeStruct((_MAXLEN, _DIM // 8, _NW, 8, _BBLK),
                                  jnp.float32),
    mesh=_mesh,
    compiler_params=pltpu.CompilerParams(use_tc_tiling_on_sc=False,
                                         needs_layout_passes=False),
    scratch_types=[
        pltpu.VMEM((_MAXLEN, _BBLK), jnp.int32),
        pltpu.VMEM((_MAXLEN, _DIM), jnp.float32),
        pltpu.VMEM((_BBLK, _DIM), jnp.float32),
        pltpu.VMEM((_BBLK, _DIM), jnp.float32),
        pltpu.VMEM((_DIM // 8, 8, _BBLK), jnp.float32),
        pltpu.VMEM((_DIM // 8, 8, _BBLK), jnp.float32),
        pltpu.SemaphoreType.DMA,
        pltpu.SemaphoreType.DMA,
        pltpu.SemaphoreType.DMA,
    ],
)
def _embed_kernel(xT_hbm, tok_hbm, pos_hbm, out_hbm,
                  idx_v, pos_v, rows0, rows1, obuf0, obuf1, sg0, sg1, sw):
    wid = lax.axis_index("s") * 2 + lax.axis_index("c")

    pltpu.sync_copy(xT_hbm.at[:, pl.ds(wid * _BBLK, _BBLK)], idx_v)
    pltpu.sync_copy(pos_hbm, pos_v)

    riota = lax.iota(jnp.int32, 16)
    rvecs = [riota + (16 * bl) for bl in range(_BBLK // 16)]

    def fire(m, rows, sem):
        pltpu.async_copy(tok_hbm.at[idx_v.at[m]], rows, sem)

    def drain_gather(rows, sem):
        pltpu.make_async_copy(tok_hbm.at[pl.ds(0, _BBLK)], rows, sem).wait()

    def drain_write(obuf):
        pltpu.make_async_copy(out_hbm.at[0, :, 0], obuf, sw).wait()

    def transpose_add(m, rows, obuf):
        p0 = pos_v[m, pl.ds(0, 16)]
        p1 = pos_v[m, pl.ds(16, 16)]

        def rbody(j, carry):
            for u in range(4):
                r = j * 4 + u
                plsc.addupdate(rows.at[r, pl.ds(0, 16)], p0)
                plsc.addupdate(rows.at[r, pl.ds(16, 16)], p1)
            return carry
        lax.fori_loop(0, _BBLK // 4, rbody, 0)

        def dbody(d, carry):
            dvec = jnp.full((16,), d, jnp.int32)
            for bl in range(_BBLK // 16):
                v = plsc.load_gather(rows, [rvecs[bl], dvec])
                obuf[d // 8, d % 8, pl.ds(bl * 16, 16)] = v
            return carry
        lax.fori_loop(0, _DIM, dbody, 0)

    def write(m, obuf):
        pltpu.async_copy(obuf, out_hbm.at[m, :, wid], sw)

    fire(0, rows0, sg0)

    def pair(i, carry):
        m0 = 2 * i
        m1 = m0 + 1

        fire(m1, rows1, sg1)
        drain_gather(rows0, sg0)

        @pl.when(i > 0)
        def _():
            drain_write(obuf0)

        transpose_add(m0, rows0, obuf0)
        write(m0, obuf0)

        @pl.when(m1 + 1 < _MAXLEN)
        def _():
            fire(m1 + 1, rows0, sg0)

        drain_gather(rows1, sg1)

        @pl.when(i > 0)
        def _():
            drain_write(obuf1)

        transpose_add(m1, rows1, obuf1)
        write(m1, obuf1)
        return carry

    lax.fori_loop(0, _MAXLEN // 2, pair, 0)
    drain_write(obuf0)
    drain_write(obuf1)


def kernel(x, token_table, pos_table):
    xT = x.astype(jnp.int32).T
    o5 = _embed_kernel(xT, token_table, pos_table)
    out = jnp.transpose(o5, (2, 4, 0, 1, 3))
    return out.reshape(_BATCH, _MAXLEN, _DIM)


# speedup vs baseline: 1.3250x; 1.3250x over previous
import functools

import jax
import jax.numpy as jnp
from jax import lax
from jax.experimental import pallas as pl
from jax.experimental.pallas import tpu as pltpu
from jax.experimental.pallas import tpu_sc as plsc

_VOCAB = 1000000
_MAXLEN = 200
_DIM = 32
_BATCH = 4096
_NW = 32
_BBLK = _BATCH // _NW
_RING = 8

_mesh = plsc.VectorSubcoreMesh(core_axis_name="c", subcore_axis_name="s")


@functools.partial(
    pl.kernel,
    out_type=jax.ShapeDtypeStruct((_MAXLEN, _DIM // 8, _NW, 8, _BBLK),
                                  jnp.float32),
    mesh=_mesh,
    compiler_params=pltpu.CompilerParams(use_tc_tiling_on_sc=False,
                                         needs_layout_passes=False),
    scratch_types=(
        [pltpu.VMEM((_MAXLEN, _BBLK), jnp.int32),
         pltpu.VMEM((_MAXLEN, _DIM), jnp.float32)]
        + [pltpu.VMEM((_BBLK, _DIM), jnp.float32)] * _RING
        + [pltpu.VMEM((_DIM // 8, 8, _BBLK), jnp.float32)] * 2
        + [pltpu.SemaphoreType.DMA, pltpu.SemaphoreType.DMA]
    ),
)
def _embed_kernel(xT_hbm, tok_hbm, pos_hbm, out_hbm,
                  idx_v, pos_v, r0, r1, r2, r3, r4, r5, r6, r7,
                  obuf0, obuf1, sg, sw):
    ring = (r0, r1, r2, r3, r4, r5, r6, r7)
    obufs = (obuf0, obuf1)
    wid = lax.axis_index("s") * 2 + lax.axis_index("c")

    pltpu.sync_copy(xT_hbm.at[:, pl.ds(wid * _BBLK, _BBLK)], idx_v)
    pltpu.sync_copy(pos_hbm, pos_v)

    riota = lax.iota(jnp.int32, 16)
    rvecs = [riota + (16 * bl) for bl in range(_BBLK // 16)]

    def fire(m, rows):
        pltpu.async_copy(tok_hbm.at[idx_v.at[m]], rows, sg)

    def drain_gather(rows):
        pltpu.make_async_copy(tok_hbm.at[pl.ds(0, _BBLK)], rows, sg).wait()

    def drain_write(obuf):
        pltpu.make_async_copy(out_hbm.at[0, :, 0], obuf, sw).wait()

    def transpose_add(m, rows, obuf):
        p0 = pos_v[m, pl.ds(0, 16)]
        p1 = pos_v[m, pl.ds(16, 16)]

        @plsc.parallel_loop(0, _BBLK, step=4, unroll=2)
        def _(r):
            for u in range(4):
                plsc.addupdate(rows.at[r + u, pl.ds(0, 16)], p0)
                plsc.addupdate(rows.at[r + u, pl.ds(16, 16)], p1)

        @plsc.parallel_loop(0, _DIM, step=1, unroll=4)
        def _(d):
            dvec = jnp.full((16,), d, jnp.int32)
            for bl in range(_BBLK // 16):
                v = plsc.load_gather(rows, [rvecs[bl], dvec])
                obuf[d // 8, d % 8, pl.ds(bl * 16, 16)] = v

    def write(m, obuf):
        pltpu.async_copy(obuf, out_hbm.at[m, :, wid], sw)

    for ph in range(_RING):
        fire(ph, ring[ph])

    def group(g, carry):
        for ph in range(_RING):
            m = g * _RING + ph
            drain_gather(ring[ph])

            @pl.when(m >= 2)
            def _():
                drain_write(obufs[ph % 2])

            transpose_add(m, ring[ph], obufs[ph % 2])
            write(m, obufs[ph % 2])

            @pl.when(m + _RING < _MAXLEN)
            def _():
                fire(m + _RING, ring[ph])
        return carry

    lax.fori_loop(0, _MAXLEN // _RING, group, 0)
    drain_write(obuf0)
    drain_write(obuf1)


def kernel(x, token_table, pos_table):
    xT = x.astype(jnp.int32).T
    o5 = _embed_kernel(xT, token_table, pos_table)
    out = jnp.transpose(o5, (2, 4, 0, 1, 3))
    return out.reshape(_BATCH, _MAXLEN, _DIM)
